# baseline (speedup 1.0000x reference)
import functools
import math

import jax
import jax.numpy as jnp
from jax.experimental import pallas as pl
from jax.experimental.pallas import tpu as pltpu


def _moe_kernel(xa_ref, xc_ref, rw_ref, w1a_ref, w1b_ref, b1_ref,
                w2a_ref, w2b_ref, b2_ref, out_ref,
                h_ref, w2s_ref, w_ref, xb_ref, *, n_chunk, ffn_tile,
                expert_col, tm):
    half = n_chunk // 2
    sub = ffn_tile // 2
    f = pl.program_id(0)
    dot1 = half
    dot2 = n_chunk + 1

    @pl.when(f == 0)
    def _router():
        xb_ref[:tm // 2, :] = xa_ref[...].astype(jnp.bfloat16)
        xb_ref[tm // 2:, :] = xc_ref[...].astype(jnp.bfloat16)
        scores = jax.lax.dot_general(
            xb_ref[...], rw_ref[...], (((1,), (1,)), ((), ())),
            preferred_element_type=jnp.float32)
        m = jnp.max(scores, axis=1, keepdims=True)
        p = jnp.exp(scores - m)
        denom = jnp.sum(p, axis=1, keepdims=True)
        w_ref[...] = p[:, expert_col:expert_col + 1] / denom

    @pl.when((f != dot1) & (f != dot2))
    def _hidden():
        slot = jnp.where(f < dot1, f, f - dot1 - 1) % half
        for i, (w1r, w2r) in enumerate(((w1a_ref, w2a_ref),
                                        (w1b_ref, w2b_ref))):
            h = jax.lax.dot_general(
                xb_ref[...], w1r[0].astype(jnp.bfloat16),
                (((1,), (1,)), ((), ())),
                preferred_element_type=jnp.float32)
            h = h + b1_ref[:, pl.ds(i * sub, sub)]
            h = 0.5 * h * (1.0 + jax.lax.erf(h * (1.0 / math.sqrt(2.0))))
            off = slot * ffn_tile + i * sub
            h_ref[:, pl.ds(off, sub)] = h.astype(jnp.bfloat16)
            w2s_ref[:, pl.ds(off, sub)] = w2r[0].astype(jnp.bfloat16)

    @pl.when(f == dot1)
    def _reduce1():
        out_ref[...] = jax.lax.dot_general(
            h_ref[...], w2s_ref[...], (((1,), (1,)), ((), ())),
            preferred_element_type=jnp.float32)

    @pl.when(f == dot2)
    def _reduce2():
        acc = jax.lax.dot_general(
            h_ref[...], w2s_ref[...], (((1,), (1,)), ((), ())),
            preferred_element_type=jnp.float32)
        out_ref[...] = (out_ref[...] + acc + b2_ref[...]) * w_ref[...]


def kernel(x, router_w, W1, b1, W2, b2):
    B_, S_, H_ = x.shape
    E_, FFN_, _ = W1.shape
    T = B_ * S_
    eid = E_ - 1
    x_flat = x.reshape(T, H_)
    rw = router_w.astype(jnp.bfloat16)
    b1e = b1[eid].reshape(1, FFN_)
    b2e = b2[eid].reshape(1, H_)

    TM = 2048
    FK = 512
    n_c = FFN_ // FK
    half = n_c // 2
    last_c = n_c - 1
    sub = FK // 2

    def chunk_idx(f):
        return jnp.clip(jnp.where(f < half, f, f - 1), 0, last_c)

    out = pl.pallas_call(
        functools.partial(_moe_kernel, n_chunk=n_c, ffn_tile=FK,
                          expert_col=eid, tm=TM),
        grid=(n_c + 2,),
        in_specs=[
            pl.BlockSpec((TM // 2, H_), lambda f: (0, 0)),
            pl.BlockSpec((TM // 2, H_), lambda f: (1, 0)),
            pl.BlockSpec((E_, H_), lambda f: (0, 0)),
            pl.BlockSpec((1, sub, H_),
                         lambda f: (eid, 2 * chunk_idx(f), 0)),
            pl.BlockSpec((1, sub, H_),
                         lambda f: (eid, 2 * chunk_idx(f) + 1, 0)),
            pl.BlockSpec((1, FK),
                         lambda f: (0, chunk_idx(f))),
            pl.BlockSpec((1, H_, sub),
                         lambda f: (eid, 0, 2 * chunk_idx(f))),
            pl.BlockSpec((1, H_, sub),
                         lambda f: (eid, 0, 2 * chunk_idx(f) + 1)),
            pl.BlockSpec((1, H_), lambda f: (0, 0)),
        ],
        out_specs=pl.BlockSpec((TM, H_), lambda f: (0, 0)),
        out_shape=jax.ShapeDtypeStruct((T, H_), jnp.float32),
        scratch_shapes=[pltpu.VMEM((TM, FFN_ // 2), jnp.bfloat16),
                        pltpu.VMEM((H_, FFN_ // 2), jnp.bfloat16),
                        pltpu.VMEM((TM, 1), jnp.float32),
                        pltpu.VMEM((TM, H_), jnp.bfloat16)],
        compiler_params=pltpu.CompilerParams(
            dimension_semantics=("arbitrary",)),
    )(x_flat, x_flat, rw, W1, W1, b1e, W2, W2, b2e)
    return out.reshape(B_, S_, H_)


# speedup vs baseline: 1.0354x; 1.0354x over previous
import functools
import math

import jax
import jax.numpy as jnp
from jax.experimental import pallas as pl
from jax.experimental.pallas import tpu as pltpu


def _moe_kernel(x_ref, rw_ref, w1_ref, b1_ref, w2_ref, b2_ref, out_ref,
                h_ref, w2b_ref, w_ref, xb_ref, *, n_chunk, ffn_tile,
                expert_col):
    half = n_chunk // 2
    f = pl.program_id(0)
    dot1 = half
    dot2 = n_chunk + 1

    @pl.when(f == 0)
    def _router():
        xb_ref[...] = x_ref[...].astype(jnp.bfloat16)
        scores = jax.lax.dot_general(
            xb_ref[...], rw_ref[...], (((1,), (1,)), ((), ())),
            preferred_element_type=jnp.float32)
        m = jnp.max(scores, axis=1, keepdims=True)
        p = jnp.exp(scores - m)
        denom = jnp.sum(p, axis=1, keepdims=True)
        w_ref[...] = p[:, expert_col:expert_col + 1] / denom

    @pl.when((f != dot1) & (f != dot2))
    def _hidden():
        slot = jnp.where(f < dot1, f, f - dot1 - 1) % half
        h = jax.lax.dot_general(
            xb_ref[...], w1_ref[0].astype(jnp.bfloat16),
            (((1,), (1,)), ((), ())),
            preferred_element_type=jnp.float32)
        h = h + b1_ref[...]
        h = 0.5 * h * (1.0 + jax.lax.erf(h * (1.0 / math.sqrt(2.0))))
        h_ref[:, pl.ds(slot * ffn_tile, ffn_tile)] = h.astype(jnp.bfloat16)
        w2b_ref[:, pl.ds(slot * ffn_tile, ffn_tile)] = (
            w2_ref[0].astype(jnp.bfloat16))

    @pl.when(f == dot1)
    def _reduce1():
        out_ref[...] = jax.lax.dot_general(
            h_ref[...], w2b_ref[...], (((1,), (1,)), ((), ())),
            preferred_element_type=jnp.float32)

    @pl.when(f == dot2)
    def _reduce2():
        acc = jax.lax.dot_general(
            h_ref[...], w2b_ref[...], (((1,), (1,)), ((), ())),
            preferred_element_type=jnp.float32)
        out_ref[...] = (out_ref[...] + acc + b2_ref[...]) * w_ref[...]


def kernel(x, router_w, W1, b1, W2, b2):
    B_, S_, H_ = x.shape
    E_, FFN_, _ = W1.shape
    T = B_ * S_
    eid = E_ - 1
    x_flat = x.reshape(T, H_)
    rw = router_w.astype(jnp.bfloat16)
    b1e = b1[eid].reshape(1, FFN_)
    b2e = b2[eid].reshape(1, H_)

    TM = 2048
    FK = 512
    n_c = FFN_ // FK
    half = n_c // 2
    last_c = n_c - 1

    def chunk_idx(f):
        return jnp.clip(jnp.where(f < half, f, f - 1), 0, last_c)

    out = pl.pallas_call(
        functools.partial(_moe_kernel, n_chunk=n_c, ffn_tile=FK,
                          expert_col=eid),
        grid=(n_c + 2,),
        in_specs=[
            pl.BlockSpec((TM, H_), lambda f: (0, 0)),
            pl.BlockSpec((E_, H_), lambda f: (0, 0)),
            pl.BlockSpec((1, FK, H_),
                         lambda f: (eid, chunk_idx(f), 0)),
            pl.BlockSpec((1, FK),
                         lambda f: (0, chunk_idx(f))),
            pl.BlockSpec((1, H_, FK),
                         lambda f: (eid, 0, chunk_idx(f))),
            pl.BlockSpec((1, H_), lambda f: (0, 0)),
        ],
        out_specs=pl.BlockSpec((TM, H_), lambda f: (0, 0)),
        out_shape=jax.ShapeDtypeStruct((T, H_), jnp.float32),
        scratch_shapes=[pltpu.VMEM((TM, FFN_ // 2), jnp.bfloat16),
                        pltpu.VMEM((H_, FFN_ // 2), jnp.bfloat16),
                        pltpu.VMEM((TM, 1), jnp.float32),
                        pltpu.VMEM((TM, H_), jnp.bfloat16)],
        compiler_params=pltpu.CompilerParams(
            dimension_semantics=("arbitrary",)),
    )(x_flat, rw, W1, b1e, W2, b2e)
    return out.reshape(B_, S_, H_)
